# Initial kernel scaffold; baseline (speedup 1.0000x reference)
import jax
import jax.numpy as jnp
from jax.experimental import pallas as pl


def kernel(x, edge_index, params):
    raise NotImplementedError("write your pallas kernel here")


# speedup vs baseline: 11.3039x; 11.3039x over previous
import functools

import jax
import jax.numpy as jnp
from jax import lax
from jax.experimental import pallas as pl
from jax.experimental.pallas import tpu as pltpu
from jax.experimental.pallas import tpu_sc as plsc

N = 10000
E = 320000
NCORE = 2
NSUB = 16
NW = NCORE * NSUB
HALF = E // NCORE
CHUNK = 80
MAXCH = 131
EPAD = 4020 * CHUNK
NPAD = 10240
EXBASE = 10048
SINKBASE = 10112
RPS = NPAD // NSUB

W_BY_FO = {8: 432, 16: 432, 32: 384, 64: 320}


@functools.lru_cache(maxsize=None)
def _sc_agg(fo):
    cp = max(16, fo)
    w = W_BY_FO[fo]
    nwin = -(-HALF // w)
    q, r = divmod(nwin, 16)
    size_hi = (q + 1) * w
    size_lo = q * w
    mesh = plsc.VectorSubcoreMesh(core_axis_name="c", subcore_axis_name="s")

    def body(m_hbm, src80, dst80, zeros_hbm, out_hbm,
             src_v, dst_v, prev_v, rows0, rows1, rb0, rb1, agg_sh,
             sem0, sem1):
        cid = lax.axis_index("c")
        sid = lax.axis_index("s")
        wid = cid * NSUB + sid
        t = sid
        lo_l = jnp.minimum(t, r) * size_hi + jnp.maximum(t - r, 0) * size_lo
        span = jnp.minimum(lo_l + jnp.where(t < r, size_hi, size_lo),
                           HALF) - lo_l
        lo = cid * HALF + lo_l
        row0 = lo // CHUNK
        ofs = lo - row0 * CHUNK
        end = ofs + span
        nch = (end + CHUNK - 1) // CHUNK

        pltpu.sync_copy(src80.at[pl.ds(row0, MAXCH)], src_v)
        pltpu.sync_copy(dst80.at[pl.ds(row0, MAXCH)], dst_v)

        @pl.when(t > 0)
        def _():
            pltpu.sync_copy(dst80.at[(lo - 16) // CHUNK], prev_v)

        pltpu.sync_copy(zeros_hbm, agg_sh.at[pl.ds(sid * RPS, RPS)])

        colp = (lo - 16) - ((lo - 16) // CHUNK) * CHUNK
        d_prev = jnp.int32(0)
        for g in range(5):
            pv = prev_v[pl.ds(g * 16, 16)]
            d_prev = jnp.where(colp == g * 16, pv[15], d_prev)
        dp = jnp.full((16,), d_prev, jnp.int32)
        ex_row = jnp.full((16,), EXBASE + wid, jnp.int32)
        sink = jnp.full((16,), SINKBASE + wid, jnp.int32)
        ofs_v = jnp.full((16,), ofs, jnp.int32)
        lnum = jnp.int32(0)
        for ch in (0, 1):
            for g in range(5):
                gl = lax.iota(jnp.int32, 16) + jnp.full((16,),
                                                        ch * CHUNK + g * 16,
                                                        jnp.int32)
                v = dst_v[ch, pl.ds(g * 16, 16)]
                match = jnp.logical_and(v == dp, gl >= ofs_v)
                cnt = plsc.all_reduce_population_count(match)
                lnum = lnum + cnt[0]
        lnum = jnp.where(t > 0, lnum, 0)
        cut_v = jnp.full((16,), ofs + lnum, jnp.int32)
        for ch in (0, 1):
            for g in range(5):
                gl = lax.iota(jnp.int32, 16) + jnp.full((16,),
                                                        ch * CHUNK + g * 16,
                                                        jnp.int32)
                v = dst_v[ch, pl.ds(g * 16, 16)]
                v = jnp.where(gl < ofs_v, sink,
                              jnp.where(gl < cut_v, ex_row, v))
                dst_v[ch, pl.ds(g * 16, 16)] = v
        tc = nch - 1
        end_v = jnp.full((16,), end, jnp.int32)
        base_v = jnp.full((16,), tc * CHUNK, jnp.int32)
        for g in range(5):
            gl = (lax.iota(jnp.int32, 16) + base_v
                  + jnp.full((16,), g * 16, jnp.int32))
            v = dst_v[tc, pl.ds(g * 16, 16)]
            dst_v[tc, pl.ds(g * 16, 16)] = jnp.where(gl < end_v, v, sink)

        plsc.subcore_barrier()

        def issue(c, buf, sem):
            pltpu.async_copy(m_hbm.at[src_v.at[c]], buf, sem)

        def drain_scatter(c, buf, sem):
            pltpu.make_async_copy(m_hbm.at[src_v.at[c]], buf, sem).wait()
            pltpu.sync_copy(buf, agg_sh.at[dst_v.at[c]], add=True)

        issue(0, rows0, sem0)

        def step(c, carry):
            nxt = c + 1
            @pl.when((nxt < nch) & (nxt % 2 == 0))
            def _():
                issue(nxt, rows0, sem0)
            @pl.when((nxt < nch) & (nxt % 2 == 1))
            def _():
                issue(nxt, rows1, sem1)
            @pl.when(c % 2 == 0)
            def _():
                drain_scatter(c, rows0, sem0)
            @pl.when(c % 2 == 1)
            def _():
                drain_scatter(c, rows1, sem1)
            return carry

        lax.fori_loop(0, nch, step, 0)
        plsc.subcore_barrier()

        @pl.when(lnum > 0)
        def _():
            pltpu.sync_copy(agg_sh.at[d_prev], rb0)
            pltpu.sync_copy(agg_sh.at[EXBASE + wid], rb1)
            for g in range(cp // 16):
                rb0[pl.ds(g * 16, 16)] = (rb0[pl.ds(g * 16, 16)]
                                          + rb1[pl.ds(g * 16, 16)])
            pltpu.sync_copy(rb0, agg_sh.at[d_prev])

        plsc.subcore_barrier()
        pltpu.sync_copy(agg_sh.at[pl.ds(sid * RPS, RPS)],
                        out_hbm.at[cid, pl.ds(sid * RPS, RPS)])

    return pl.kernel(
        body,
        out_type=jax.ShapeDtypeStruct((NCORE, NPAD, cp), jnp.float32),
        mesh=mesh,
        compiler_params=pltpu.CompilerParams(use_tc_tiling_on_sc=False,
                                             needs_layout_passes=False),
        scratch_types=[
            pltpu.VMEM((MAXCH, CHUNK), jnp.int32),
            pltpu.VMEM((MAXCH, CHUNK), jnp.int32),
            pltpu.VMEM((CHUNK,), jnp.int32),
            pltpu.VMEM((CHUNK, cp), jnp.float32),
            pltpu.VMEM((CHUNK, cp), jnp.float32),
            pltpu.VMEM((cp,), jnp.float32),
            pltpu.VMEM((cp,), jnp.float32),
            pltpu.VMEM_SHARED((NPAD, cp), jnp.float32),
            pltpu.SemaphoreType.DMA,
            pltpu.SemaphoreType.DMA,
        ],
    )


def _bn(t):
    m = jnp.mean(t, axis=0, keepdims=True)
    v = jnp.mean(jnp.square(t - m), axis=0, keepdims=True)
    return (t - m) / jnp.sqrt(v + 1e-5)


def _dot(a, b):
    return jnp.dot(a, b, preferred_element_type=jnp.float32)


def _mm2(h, ws, wnp):
    def f(h_ref, ws_ref, wn_ref, s_ref, m_ref):
        h_ = h_ref[...]
        s_ref[...] = _dot(h_, ws_ref[...])
        m_ref[...] = _dot(h_, wn_ref[...])
    return pl.pallas_call(
        f,
        out_shape=(jax.ShapeDtypeStruct((N, ws.shape[1]), jnp.float32),
                   jax.ShapeDtypeStruct((N, wnp.shape[1]), jnp.float32)),
    )(h, ws, wnp)


def _mm2_ds(h, ws, wnp, wds):
    def f(h_ref, ws_ref, wn_ref, wd_ref, s_ref, m_ref, r_ref):
        h_ = h_ref[...]
        s_ref[...] = _dot(h_, ws_ref[...])
        m_ref[...] = _dot(h_, wn_ref[...])
        r_ref[...] = _bn(_dot(h_, wd_ref[...]))
    return pl.pallas_call(
        f,
        out_shape=(jax.ShapeDtypeStruct((N, ws.shape[1]), jnp.float32),
                   jax.ShapeDtypeStruct((N, wnp.shape[1]), jnp.float32),
                   jax.ShapeDtypeStruct((N, wds.shape[1]), jnp.float32)),
    )(h, ws, wnp, wds)


def _finish(s, part, res=None):
    fo = s.shape[1]

    def f_nores(s_ref, p_ref, o_ref):
        t = (p_ref[0, :N, :fo] + p_ref[1, :N, :fo]) + s_ref[...]
        o_ref[...] = jnp.maximum(_bn(t), 0.0)

    def f_res(s_ref, p_ref, r_ref, o_ref):
        t = (p_ref[0, :N, :fo] + p_ref[1, :N, :fo]) + s_ref[...]
        o_ref[...] = jnp.maximum(_bn(t) + r_ref[...], 0.0)

    out = jax.ShapeDtypeStruct((N, fo), jnp.float32)
    if res is None:
        return pl.pallas_call(f_nores, out_shape=out)(s, part)
    return pl.pallas_call(f_res, out_shape=out)(s, part, res)


def _head(h, w, b):
    def f(h_ref, w_ref, b_ref, o_ref):
        o_ref[...] = _dot(h_ref[...], w_ref[...]) + b_ref[...]
    return pl.pallas_call(
        f, out_shape=jax.ShapeDtypeStruct((N, w.shape[1]), jnp.float32),
    )(h, w, b.reshape(1, -1))


def kernel(x, edge_index, params):
    p = params
    src32 = edge_index[0].astype(jnp.int32)
    dst32 = edge_index[1].astype(jnp.int32)
    perm = jnp.argsort(dst32, stable=True)
    ssrc = jnp.pad(src32[perm], (0, EPAD - E)).reshape(-1, CHUNK)
    sdst = jnp.pad(dst32[perm], (0, EPAD - E)).reshape(-1, CHUNK)

    def agg(m):
        fo = m.shape[1]
        cp = max(16, fo)
        if fo < cp:
            m = jnp.pad(m, ((0, 0), (0, cp - fo)))
        zeros = jnp.zeros((RPS, cp), jnp.float32)
        return _sc_agg(fo)(m, ssrc, sdst, zeros)

    def pad_wn(wn):
        fo = wn.shape[1]
        cp = max(16, fo)
        return jnp.pad(wn, ((0, 0), (0, cp - fo))) if cp > fo else wn

    def conv_pre(h, name, res=None):
        s, m = _mm2(h, p[name + "_s"], pad_wn(p[name + "_n"]))
        return _finish(s, agg(m), res)

    def block_b1(h):
        o = conv_pre(h, "b1c1")
        return conv_pre(o, "b1c2", res=h)

    def block_pre(h, b):
        s, m, res = _mm2_ds(h, p[b + "c1_s"], pad_wn(p[b + "c1_n"]),
                            p[b + "ds_s"])
        o = _finish(s, agg(m))
        s2, m2 = _mm2(o, p[b + "c2_s"], pad_wn(p[b + "c2_n"]))
        return _finish(s2, agg(m2), res)

    out_p1 = conv_pre(x, "conv0")
    out = conv_pre(out_p1, "conv1")
    out_b1 = block_b1(out)
    out = conv_pre(out_b1, "conv2")
    out_b2 = block_pre(out, "b2")
    out = conv_pre(out_b2, "conv3")
    out_b3 = block_pre(out, "b3")
    out = conv_pre(out_b3, "conv4")
    out = block_pre(out, "b4")
    out = conv_pre(out, "tr4")
    out = block_pre(jnp.concatenate([out, out_b3], axis=1), "b5")
    out = conv_pre(out, "tr5")
    out = block_pre(jnp.concatenate([out, out_b2], axis=1), "b6")
    out = conv_pre(out, "tr6")
    out = block_pre(jnp.concatenate([out, out_b1], axis=1), "b7")
    out = conv_pre(out, "tr7")
    out = block_pre(jnp.concatenate([out, out_p1], axis=1), "b8")
    return _head(out, p["final_w"], p["final_b"])
